# baseline (speedup 1.0000x reference)
import functools

import jax
import jax.numpy as jnp
from jax import lax
from jax.experimental import pallas as pl
from jax.experimental.pallas import tpu as pltpu
from jax.experimental.pallas import tpu_sc as plsc

N_NODES = 10000
N_EDGES = 320000
E2 = N_EDGES // 2
D = 128


def _node_proj(nfeat, W_s, W_d):
    NB = 2000

    def body(nf_ref, ws_ref, wd_ref, ps_ref, pd_ref):
        x = nf_ref[...]
        dn = (((1,), (1,)), ((), ()))
        ps_ref[...] = lax.dot_general(x, ws_ref[...], dn,
                                      preferred_element_type=jnp.float32)
        pd_ref[...] = lax.dot_general(x, wd_ref[...], dn,
                                      preferred_element_type=jnp.float32)

    return pl.pallas_call(
        body,
        grid=(N_NODES // NB,),
        in_specs=[
            pl.BlockSpec((NB, D), lambda i: (i, 0)),
            pl.BlockSpec((D, D), lambda i: (0, 0)),
            pl.BlockSpec((D, D), lambda i: (0, 0)),
        ],
        out_specs=[
            pl.BlockSpec((NB, D), lambda i: (i, 0)),
            pl.BlockSpec((NB, D), lambda i: (i, 0)),
        ],
        out_shape=[jax.ShapeDtypeStruct((N_NODES, D), jnp.float32)] * 2,
    )(nfeat, W_s, W_d)


@functools.cache
def _make_gather_pack(n_pairs=E2, row_off=0):
    info = plsc.get_sparse_core_info()
    NC, NS, L = info.num_cores, info.num_subcores, info.num_lanes
    NW = NC * NS
    pw = n_pairs // NW
    CH = 40
    n_ch = pw // CH
    assert pw % CH == 0 and CH % 8 == 0 and n_ch >= 6
    NSL = 4
    NG = D // L

    mesh = plsc.VectorSubcoreMesh(core_axis_name="c", subcore_axis_name="s")

    @functools.partial(
        pl.kernel,
        mesh=mesh,
        out_type=jax.ShapeDtypeStruct((n_pairs, D), jnp.int32),
        scratch_types=(
            [pltpu.VMEM((pw,), jnp.int32)] * 4
            + [pltpu.VMEM((CH, D), jnp.float32)] * (4 * NSL)
            + [pltpu.VMEM((CH, D), jnp.int32)] * NSL
            + [pltpu.SemaphoreType.DMA] * (2 * NSL)
        ),
    )
    def gather_pack(ps_hbm, pd_hbm, src_hbm, dst_hbm, out_hbm, *refs):
        isa, ida, isb, idb = refs[0:4]
        rows = refs[4:4 + 4 * NSL]
        pk = refs[4 + 4 * NSL:4 + 5 * NSL]
        sem_g = refs[4 + 5 * NSL:4 + 6 * NSL]
        sem_w = refs[4 + 6 * NSL:4 + 7 * NSL]

        wid = lax.axis_index("s") * NC + lax.axis_index("c")
        base = wid * pw
        ebase = row_off + base
        pltpu.sync_copy(src_hbm.at[pl.ds(ebase, pw)], isa)
        pltpu.sync_copy(dst_hbm.at[pl.ds(ebase, pw)], ida)
        pltpu.sync_copy(src_hbm.at[pl.ds(E2 + ebase, pw)], isb)
        pltpu.sync_copy(dst_hbm.at[pl.ds(E2 + ebase, pw)], idb)

        def g_start(c, b):
            off = c * CH
            r = rows[4 * b:4 * b + 4]
            for tab, idx, dstb in ((ps_hbm, isa, r[0]), (pd_hbm, ida, r[1]),
                                   (ps_hbm, isb, r[2]), (pd_hbm, idb, r[3])):
                pltpu.make_async_copy(tab.at[idx.at[pl.ds(off, CH)]],
                                      dstb, sem_g[b]).start()

        def g_wait(b):
            for k in range(4):
                pltpu.make_async_copy(ps_hbm.at[isa.at[pl.ds(0, CH)]],
                                      rows[4 * b + k], sem_g[b]).wait()

        def wb_start(c, b):
            pltpu.make_async_copy(pk[b],
                                  out_hbm.at[pl.ds(base + c * CH, CH)],
                                  sem_w[b]).start()

        def wb_wait(b):
            pltpu.make_async_copy(pk[b],
                                  out_hbm.at[pl.ds(base, CH)],
                                  sem_w[b]).wait()

        def addpack(b):
            ras, rad, rbs, rbd = rows[4 * b:4 * b + 4]
            pkb = pk[b]
            c7 = jnp.int32(0x7FFF)
            c1 = jnp.int32(1)

            def row_op(r, carry):
                for j in range(NG):
                    sl = (r, pl.ds(j * L, L))
                    sa = ras[sl] + rad[sl]
                    sb = rbs[sl] + rbd[sl]
                    ua = lax.bitcast_convert_type(sa, jnp.int32)
                    ub = lax.bitcast_convert_type(sb, jnp.int32)
                    ra = lax.shift_right_logical(
                        ua + c7 + (lax.shift_right_logical(ua, 16) & c1), 16)
                    rb = lax.shift_right_logical(
                        ub + c7 + (lax.shift_right_logical(ub, 16) & c1), 16)
                    pkb[sl] = ra | (rb << 16)
                return carry

            lax.fori_loop(0, CH, row_op, 0)

        g_start(0, 0)
        g_start(1, 1)
        g_start(2, 2)
        g_wait(0); addpack(0); wb_start(0, 0)
        g_start(3, 3)
        g_wait(1); addpack(1); wb_start(1, 1)

        def main_body(i, carry):
            c0 = 2 + i * NSL
            for b in range(NSL):
                c = c0 + b
                sg = b
                sc = (2 + b) % NSL
                wb_wait(sg)
                g_start(c + 2, sg)
                g_wait(sc); addpack(sc); wb_start(c, sc)
            return carry

        k = (n_ch - 4) // NSL
        lax.fori_loop(0, k, main_body, 0)

        for c in range(2 + 4 * k, n_ch):
            if c + 2 < n_ch:
                wb_wait((c + 2) % NSL)
                g_start(c + 2, (c + 2) % NSL)
            g_wait(c % NSL); addpack(c % NSL); wb_start(c, c % NSL)
        for c in range(n_ch - 4, n_ch):
            wb_wait(c % NSL)

    return gather_pack


def _edge_mlp(efeat, gpk, W_e, W_f, b1, b_f, ln_g, ln_b, i_off=0, prev=None):
    EB = 8000
    NB2 = E2 // EB
    nblk = gpk.shape[0] // EB

    def body(e_ref, g_ref, we_ref, wf_ref, b1_ref, bf_ref, lng_ref, lnb_ref,
             *rest):
        o_ref = rest[-1]
        j = pl.program_id(1)
        e = e_ref[...]
        dn = (((1,), (1,)), ((), ()))
        h = lax.dot_general(e.astype(jnp.bfloat16),
                            we_ref[...].astype(jnp.bfloat16), dn,
                            preferred_element_type=jnp.float32)
        u = g_ref[...]
        g = lax.bitcast_convert_type((u >> (16 * j)) << 16, jnp.float32)
        h = h + g + b1_ref[...]
        h = h * jax.nn.sigmoid(h)
        o = lax.dot_general(h.astype(jnp.bfloat16),
                            wf_ref[...].astype(jnp.bfloat16), dn,
                            preferred_element_type=jnp.float32) + bf_ref[...]
        mu = jnp.mean(o, axis=1, keepdims=True)
        var = jnp.mean((o - mu) * (o - mu), axis=1, keepdims=True)
        o = (o - mu) * lax.rsqrt(var + 1e-5) * lng_ref[...] + lnb_ref[...]
        o_ref[...] = o + e

    vec = pl.BlockSpec((1, D), lambda i, j: (0, 0))
    blkmap = lambda i, j: (j * NB2 + i_off + i, 0)
    in_specs = [
        pl.BlockSpec((EB, D), blkmap),
        pl.BlockSpec((EB, D), lambda i, j: (i, 0)),
        pl.BlockSpec((D, D), lambda i, j: (0, 0)),
        pl.BlockSpec((D, D), lambda i, j: (0, 0)),
        vec, vec, vec, vec,
    ]
    args = [efeat, gpk, W_e, W_f, b1.reshape(1, D), b_f.reshape(1, D),
            ln_g.reshape(1, D), ln_b.reshape(1, D)]
    aliases = {}
    if prev is not None:
        in_specs.append(pl.BlockSpec(memory_space=pl.ANY))
        args.append(prev)
        aliases = {8: 0}
    return pl.pallas_call(
        body,
        grid=(nblk, 2),
        in_specs=in_specs,
        out_specs=pl.BlockSpec((EB, D), blkmap),
        out_shape=jax.ShapeDtypeStruct((N_EDGES, D), jnp.float32),
        input_output_aliases=aliases,
    )(*args)


def kernel(efeat, nfeat, edge_index, W_e, W_s, W_d, b1, W_f, b_f, ln_g, ln_b):
    src = edge_index[0]
    dst = edge_index[1]
    ps, pd = _node_proj(nfeat, W_s, W_d)
    EA = 64000
    gp1 = _make_gather_pack(EA, 0)(ps, pd, src, dst)
    gp2 = _make_gather_pack(E2 - EA, EA)(ps, pd, src, dst)
    gpk = jnp.concatenate([gp1, gp2], axis=0)
    out = _edge_mlp(efeat, gpk, W_e, W_f, b1, b_f, ln_g, ln_b)
    return (out, nfeat)


# speedup vs baseline: 1.1821x; 1.1821x over previous
import functools

import jax
import jax.numpy as jnp
from jax import lax
from jax.experimental import pallas as pl
from jax.experimental.pallas import tpu as pltpu
from jax.experimental.pallas import tpu_sc as plsc

N_NODES = 10000
N_EDGES = 320000
E2 = N_EDGES // 2
D = 128


def _node_proj(nfeat, W_s, W_d):
    NB = 2000

    def body(nf_ref, ws_ref, wd_ref, ps_ref, pd_ref):
        x = nf_ref[...]
        dn = (((1,), (1,)), ((), ()))
        ps_ref[...] = lax.dot_general(x, ws_ref[...], dn,
                                      preferred_element_type=jnp.float32)
        pd_ref[...] = lax.dot_general(x, wd_ref[...], dn,
                                      preferred_element_type=jnp.float32)

    return pl.pallas_call(
        body,
        grid=(N_NODES // NB,),
        in_specs=[
            pl.BlockSpec((NB, D), lambda i: (i, 0)),
            pl.BlockSpec((D, D), lambda i: (0, 0)),
            pl.BlockSpec((D, D), lambda i: (0, 0)),
        ],
        out_specs=[
            pl.BlockSpec((NB, D), lambda i: (i, 0)),
            pl.BlockSpec((NB, D), lambda i: (i, 0)),
        ],
        out_shape=[jax.ShapeDtypeStruct((N_NODES, D), jnp.float32)] * 2,
    )(nfeat, W_s, W_d)


@functools.cache
def _make_gather_pack(n_pairs=E2, row_off=0):
    info = plsc.get_sparse_core_info()
    NC, NS, L = info.num_cores, info.num_subcores, info.num_lanes
    NW = NC * NS
    pw = n_pairs // NW
    CH = 40
    n_ch = pw // CH
    assert pw % CH == 0 and CH % 8 == 0 and n_ch >= 6
    NSL = 4
    NG = D // L

    mesh = plsc.VectorSubcoreMesh(core_axis_name="c", subcore_axis_name="s")

    @functools.partial(
        pl.kernel,
        mesh=mesh,
        out_type=jax.ShapeDtypeStruct((n_pairs, D), jnp.int32),
        scratch_types=(
            [pltpu.VMEM((pw,), jnp.int32)] * 4
            + [pltpu.VMEM((CH, D), jnp.float32)] * (4 * NSL)
            + [pltpu.VMEM((CH, D), jnp.int32)] * NSL
            + [pltpu.SemaphoreType.DMA] * (2 * NSL)
        ),
    )
    def gather_pack(ps_hbm, pd_hbm, src_hbm, dst_hbm, out_hbm, *refs):
        isa, ida, isb, idb = refs[0:4]
        rows = refs[4:4 + 4 * NSL]
        pk = refs[4 + 4 * NSL:4 + 5 * NSL]
        sem_g = refs[4 + 5 * NSL:4 + 6 * NSL]
        sem_w = refs[4 + 6 * NSL:4 + 7 * NSL]

        wid = lax.axis_index("s") * NC + lax.axis_index("c")
        base = wid * pw
        ebase = row_off + base
        pltpu.sync_copy(src_hbm.at[pl.ds(ebase, pw)], isa)
        pltpu.sync_copy(dst_hbm.at[pl.ds(ebase, pw)], ida)
        pltpu.sync_copy(src_hbm.at[pl.ds(E2 + ebase, pw)], isb)
        pltpu.sync_copy(dst_hbm.at[pl.ds(E2 + ebase, pw)], idb)

        def g_start(c, b):
            off = c * CH
            r = rows[4 * b:4 * b + 4]
            for tab, idx, dstb in ((ps_hbm, isa, r[0]), (pd_hbm, ida, r[1]),
                                   (ps_hbm, isb, r[2]), (pd_hbm, idb, r[3])):
                pltpu.make_async_copy(tab.at[idx.at[pl.ds(off, CH)]],
                                      dstb, sem_g[b]).start()

        def g_wait(b):
            for k in range(4):
                pltpu.make_async_copy(ps_hbm.at[isa.at[pl.ds(0, CH)]],
                                      rows[4 * b + k], sem_g[b]).wait()

        def wb_start(c, b):
            pltpu.make_async_copy(pk[b],
                                  out_hbm.at[pl.ds(base + c * CH, CH)],
                                  sem_w[b]).start()

        def wb_wait(b):
            pltpu.make_async_copy(pk[b],
                                  out_hbm.at[pl.ds(base, CH)],
                                  sem_w[b]).wait()

        def addpack(b):
            ras, rad, rbs, rbd = rows[4 * b:4 * b + 4]
            pkb = pk[b]
            c7 = jnp.int32(0x7FFF)
            c1 = jnp.int32(1)

            def row_op(r, carry):
                for j in range(NG):
                    sl = (r, pl.ds(j * L, L))
                    sa = ras[sl] + rad[sl]
                    sb = rbs[sl] + rbd[sl]
                    ua = lax.bitcast_convert_type(sa, jnp.int32)
                    ub = lax.bitcast_convert_type(sb, jnp.int32)
                    ra = lax.shift_right_logical(
                        ua + c7 + (lax.shift_right_logical(ua, 16) & c1), 16)
                    rb = lax.shift_right_logical(
                        ub + c7 + (lax.shift_right_logical(ub, 16) & c1), 16)
                    pkb[sl] = ra | (rb << 16)
                return carry

            lax.fori_loop(0, CH, row_op, 0)

        g_start(0, 0)
        g_start(1, 1)
        g_start(2, 2)
        g_wait(0); addpack(0); wb_start(0, 0)
        g_start(3, 3)
        g_wait(1); addpack(1); wb_start(1, 1)

        def main_body(i, carry):
            c0 = 2 + i * NSL
            for b in range(NSL):
                c = c0 + b
                sg = b
                sc = (2 + b) % NSL
                wb_wait(sg)
                g_start(c + 2, sg)
                g_wait(sc); addpack(sc); wb_start(c, sc)
            return carry

        k = (n_ch - 4) // NSL
        lax.fori_loop(0, k, main_body, 0)

        for c in range(2 + 4 * k, n_ch):
            if c + 2 < n_ch:
                wb_wait((c + 2) % NSL)
                g_start(c + 2, (c + 2) % NSL)
            g_wait(c % NSL); addpack(c % NSL); wb_start(c, c % NSL)
        for c in range(n_ch - 4, n_ch):
            wb_wait(c % NSL)

    return gather_pack


def _edge_mlp(efeat, gpk, W_e, W_f, b1, b_f, ln_g, ln_b, i_off=0, prev=None):
    EB = 8000
    NB2 = E2 // EB
    nblk = gpk.shape[0] // EB

    def body(e_ref, g_ref, we_ref, wf_ref, b1_ref, bf_ref, lng_ref, lnb_ref,
             *rest):
        o_ref = rest[-1]
        j = pl.program_id(1)
        e = e_ref[...]
        dn = (((1,), (1,)), ((), ()))
        h = lax.dot_general(e.astype(jnp.bfloat16),
                            we_ref[...].astype(jnp.bfloat16), dn,
                            preferred_element_type=jnp.float32)
        u = g_ref[...]
        g = lax.bitcast_convert_type((u >> (16 * j)) << 16, jnp.float32)
        h = h + g + b1_ref[...]
        h = h * jax.nn.sigmoid(h)
        o = lax.dot_general(h.astype(jnp.bfloat16),
                            wf_ref[...].astype(jnp.bfloat16), dn,
                            preferred_element_type=jnp.float32) + bf_ref[...]
        mu = jnp.mean(o, axis=1, keepdims=True)
        var = jnp.mean((o - mu) * (o - mu), axis=1, keepdims=True)
        o = (o - mu) * lax.rsqrt(var + 1e-5) * lng_ref[...] + lnb_ref[...]
        o_ref[...] = o + e

    vec = pl.BlockSpec((1, D), lambda i, j: (0, 0))
    blkmap = lambda i, j: (j * NB2 + i_off + i, 0)
    in_specs = [
        pl.BlockSpec((EB, D), blkmap),
        pl.BlockSpec((EB, D), lambda i, j: (i, 0)),
        pl.BlockSpec((D, D), lambda i, j: (0, 0)),
        pl.BlockSpec((D, D), lambda i, j: (0, 0)),
        vec, vec, vec, vec,
    ]
    args = [efeat, gpk, W_e, W_f, b1.reshape(1, D), b_f.reshape(1, D),
            ln_g.reshape(1, D), ln_b.reshape(1, D)]
    aliases = {}
    if prev is not None:
        in_specs.append(pl.BlockSpec(memory_space=pl.ANY))
        args.append(prev)
        aliases = {8: 0}
    return pl.pallas_call(
        body,
        grid=(nblk, 2),
        in_specs=in_specs,
        out_specs=pl.BlockSpec((EB, D), blkmap),
        out_shape=jax.ShapeDtypeStruct((N_EDGES, D), jnp.float32),
        input_output_aliases=aliases,
    )(*args)


def kernel(efeat, nfeat, edge_index, W_e, W_s, W_d, b1, W_f, b_f, ln_g, ln_b):
    src = edge_index[0]
    dst = edge_index[1]
    ps, pd = _node_proj(nfeat, W_s, W_d)
    gpk = _make_gather_pack(E2, 0)(ps, pd, src, dst)
    out = _edge_mlp(efeat, gpk, W_e, W_f, b1, b_f, ln_g, ln_b)
    return (out, nfeat)


# speedup vs baseline: 1.2109x; 1.0244x over previous
import functools

import jax
import jax.numpy as jnp
from jax import lax
from jax.experimental import pallas as pl
from jax.experimental.pallas import tpu as pltpu
from jax.experimental.pallas import tpu_sc as plsc

N_NODES = 10000
N_EDGES = 320000
E2 = N_EDGES // 2
D = 128


def _node_proj(nfeat, W_s, W_d):
    NB = 2000

    def body(nf_ref, ws_ref, wd_ref, ps_ref, pd_ref):
        x = nf_ref[...]
        dn = (((1,), (1,)), ((), ()))
        ps_ref[...] = lax.dot_general(x, ws_ref[...], dn,
                                      preferred_element_type=jnp.float32)
        pd_ref[...] = lax.dot_general(x, wd_ref[...], dn,
                                      preferred_element_type=jnp.float32)

    return pl.pallas_call(
        body,
        grid=(N_NODES // NB,),
        in_specs=[
            pl.BlockSpec((NB, D), lambda i: (i, 0)),
            pl.BlockSpec((D, D), lambda i: (0, 0)),
            pl.BlockSpec((D, D), lambda i: (0, 0)),
        ],
        out_specs=[
            pl.BlockSpec((NB, D), lambda i: (i, 0)),
            pl.BlockSpec((NB, D), lambda i: (i, 0)),
        ],
        out_shape=[jax.ShapeDtypeStruct((N_NODES, D), jnp.float32)] * 2,
    )(nfeat, W_s, W_d)


@functools.cache
def _make_gather_pack(n_pairs=E2, row_off=0):
    info = plsc.get_sparse_core_info()
    NC, NS, L = info.num_cores, info.num_subcores, info.num_lanes
    NW = NC * NS
    pw = n_pairs // NW
    CH = 40
    n_ch = pw // CH
    assert pw % CH == 0 and CH % 8 == 0 and n_ch >= 6
    NSL = 4
    NG = D // L

    mesh = plsc.VectorSubcoreMesh(core_axis_name="c", subcore_axis_name="s")

    @functools.partial(
        pl.kernel,
        mesh=mesh,
        out_type=jax.ShapeDtypeStruct((n_pairs, D), jnp.int32),
        scratch_types=(
            [pltpu.VMEM((pw,), jnp.int32)] * 4
            + [pltpu.VMEM((CH, D), jnp.float32)] * (4 * NSL)
            + [pltpu.VMEM((CH, D), jnp.int32)] * NSL
            + [pltpu.SemaphoreType.DMA] * (2 * NSL)
        ),
    )
    def gather_pack(ps_hbm, pd_hbm, src_hbm, dst_hbm, out_hbm, *refs):
        isa, ida, isb, idb = refs[0:4]
        rows = refs[4:4 + 4 * NSL]
        pk = refs[4 + 4 * NSL:4 + 5 * NSL]
        sem_g = refs[4 + 5 * NSL:4 + 6 * NSL]
        sem_w = refs[4 + 6 * NSL:4 + 7 * NSL]

        wid = lax.axis_index("s") * NC + lax.axis_index("c")
        base = wid * pw
        ebase = row_off + base
        pltpu.sync_copy(src_hbm.at[pl.ds(ebase, pw)], isa)
        pltpu.sync_copy(dst_hbm.at[pl.ds(ebase, pw)], ida)
        pltpu.sync_copy(src_hbm.at[pl.ds(E2 + ebase, pw)], isb)
        pltpu.sync_copy(dst_hbm.at[pl.ds(E2 + ebase, pw)], idb)

        def g_start(c, b):
            off = c * CH
            r = rows[4 * b:4 * b + 4]
            for tab, idx, dstb in ((ps_hbm, isa, r[0]), (pd_hbm, ida, r[1]),
                                   (ps_hbm, isb, r[2]), (pd_hbm, idb, r[3])):
                pltpu.make_async_copy(tab.at[idx.at[pl.ds(off, CH)]],
                                      dstb, sem_g[b]).start()

        def g_wait(b):
            for k in range(4):
                pltpu.make_async_copy(ps_hbm.at[isa.at[pl.ds(0, CH)]],
                                      rows[4 * b + k], sem_g[b]).wait()

        def wb_start(c, b):
            pltpu.make_async_copy(pk[b],
                                  out_hbm.at[pl.ds(base + c * CH, CH)],
                                  sem_w[b]).start()

        def wb_wait(b):
            pltpu.make_async_copy(pk[b],
                                  out_hbm.at[pl.ds(base, CH)],
                                  sem_w[b]).wait()

        def addpack(b):
            ras, rad, rbs, rbd = rows[4 * b:4 * b + 4]
            pkb = pk[b]
            c7 = jnp.int32(0x7FFF)
            c1 = jnp.int32(1)

            def row_op(r, carry):
                for j in range(NG):
                    sl = (r, pl.ds(j * L, L))
                    sa = ras[sl] + rad[sl]
                    sb = rbs[sl] + rbd[sl]
                    ua = lax.bitcast_convert_type(sa, jnp.int32)
                    ub = lax.bitcast_convert_type(sb, jnp.int32)
                    ra = lax.shift_right_logical(
                        ua + c7 + (lax.shift_right_logical(ua, 16) & c1), 16)
                    rb = lax.shift_right_logical(
                        ub + c7 + (lax.shift_right_logical(ub, 16) & c1), 16)
                    pkb[sl] = ra | (rb << 16)
                return carry

            lax.fori_loop(0, CH, row_op, 0)

        g_start(0, 0)
        g_start(1, 1)
        g_start(2, 2)
        g_wait(0); addpack(0); wb_start(0, 0)
        g_start(3, 3)
        g_wait(1); addpack(1); wb_start(1, 1)

        def main_body(i, carry):
            c0 = 2 + i * NSL
            for b in range(NSL):
                c = c0 + b
                sg = b
                sc = (2 + b) % NSL
                wb_wait(sg)
                g_start(c + 2, sg)
                g_wait(sc); addpack(sc); wb_start(c, sc)
            return carry

        k = (n_ch - 4) // NSL
        lax.fori_loop(0, k, main_body, 0)

        for c in range(2 + 4 * k, n_ch):
            if c + 2 < n_ch:
                wb_wait((c + 2) % NSL)
                g_start(c + 2, (c + 2) % NSL)
            g_wait(c % NSL); addpack(c % NSL); wb_start(c, c % NSL)
        for c in range(n_ch - 4, n_ch):
            wb_wait(c % NSL)

    return gather_pack


def _edge_mlp(efeat, gpk, W_e, W_f, b1, b_f, ln_g, ln_b, i_off=0, prev=None):
    EB = 8000
    NB2 = E2 // EB
    nblk = gpk.shape[0] // EB

    def body(e_ref, g_ref, we_ref, wf_ref, b1_ref, bf_ref, lng_ref, lnb_ref,
             *rest):
        o_ref = rest[-1]
        j = pl.program_id(0) % 2
        e = e_ref[...]
        dn = (((1,), (1,)), ((), ()))
        h = lax.dot_general(e.astype(jnp.bfloat16),
                            we_ref[...].astype(jnp.bfloat16), dn,
                            preferred_element_type=jnp.float32)
        u = g_ref[...]
        g = lax.bitcast_convert_type((u >> (16 * j)) << 16, jnp.float32)
        h = h + g + b1_ref[...]
        h = h * jax.nn.sigmoid(h)
        o = lax.dot_general(h.astype(jnp.bfloat16),
                            wf_ref[...].astype(jnp.bfloat16), dn,
                            preferred_element_type=jnp.float32) + bf_ref[...]
        mu = jnp.mean(o, axis=1, keepdims=True)
        var = jnp.mean((o - mu) * (o - mu), axis=1, keepdims=True)
        o = (o - mu) * lax.rsqrt(var + 1e-5) * lng_ref[...] + lnb_ref[...]
        o_ref[...] = o + e

    vec = pl.BlockSpec((1, D), lambda i2: (0, 0))
    blkmap = lambda i2: ((i2 % 2) * NB2 + i_off + i2 // 2, 0)
    in_specs = [
        pl.BlockSpec((EB, D), blkmap),
        pl.BlockSpec((EB, D), lambda i2: (i2 // 2, 0)),
        pl.BlockSpec((D, D), lambda i2: (0, 0)),
        pl.BlockSpec((D, D), lambda i2: (0, 0)),
        vec, vec, vec, vec,
    ]
    args = [efeat, gpk, W_e, W_f, b1.reshape(1, D), b_f.reshape(1, D),
            ln_g.reshape(1, D), ln_b.reshape(1, D)]
    aliases = {}
    if prev is not None:
        in_specs.append(pl.BlockSpec(memory_space=pl.ANY))
        args.append(prev)
        aliases = {8: 0}
    return pl.pallas_call(
        body,
        grid=(nblk * 2,),
        in_specs=in_specs,
        out_specs=pl.BlockSpec((EB, D), blkmap),
        out_shape=jax.ShapeDtypeStruct((N_EDGES, D), jnp.float32),
        input_output_aliases=aliases,
    )(*args)


def kernel(efeat, nfeat, edge_index, W_e, W_s, W_d, b1, W_f, b_f, ln_g, ln_b):
    src = edge_index[0]
    dst = edge_index[1]
    ps, pd = _node_proj(nfeat, W_s, W_d)
    EA = 64000
    gp1 = _make_gather_pack(EA, 0)(ps, pd, src, dst)
    gp2 = _make_gather_pack(E2 - EA, EA)(ps, pd, src, dst)
    out1 = _edge_mlp(efeat, gp1, W_e, W_f, b1, b_f, ln_g, ln_b)
    out = _edge_mlp(efeat, gp2, W_e, W_f, b1, b_f, ln_g, ln_b,
                    i_off=EA // 8000, prev=out1)
    return (out, nfeat)
